# Initial kernel scaffold; baseline (speedup 1.0000x reference)
import jax
import jax.numpy as jnp
from jax.experimental import pallas as pl


def kernel(user_tensor, item_tensor, user_emb, item_emb):
    raise NotImplementedError("write your pallas kernel here")


# speedup vs baseline: 2.8439x; 2.8439x over previous
import functools

import jax
import jax.numpy as jnp
from jax import lax
from jax.experimental import pallas as pl
from jax.experimental.pallas import tpu as pltpu
from jax.experimental.pallas import tpu_sc as plsc

B = 4096
D = 32
NC = 2
NS = 16
NW = NC * NS
B_PER_W = B // NW


def _sc_gather(user_emb, item_emb, user_idx, item_idx):
    mesh = plsc.VectorSubcoreMesh(core_axis_name="c", subcore_axis_name="s")

    @functools.partial(
        pl.kernel,
        mesh=mesh,
        compiler_params=pltpu.CompilerParams(use_tc_tiling_on_sc=False),
        out_type=(
            jax.ShapeDtypeStruct((B, D), jnp.float32),
            jax.ShapeDtypeStruct((B, D), jnp.float32),
        ),
        scratch_types=[
            pltpu.VMEM((B_PER_W,), jnp.int32),
            pltpu.VMEM((B_PER_W, D), jnp.float32),
            pltpu.VMEM((B_PER_W,), jnp.int32),
            pltpu.VMEM((B_PER_W, D), jnp.float32),
            pltpu.SemaphoreType.DMA,
            pltpu.SemaphoreType.DMA,
        ],
    )
    def gather_kernel(uemb, iemb, uidx, iidx, wu_out, hi_out,
                      uidx_v, urows_v, iidx_v, irows_v, usem, isem):
        wid = lax.axis_index("s") * NC + lax.axis_index("c")
        base = wid * B_PER_W
        pltpu.sync_copy(uidx.at[pl.ds(base, B_PER_W)], uidx_v)
        pltpu.sync_copy(iidx.at[pl.ds(base, B_PER_W)], iidx_v)
        ucopy = pltpu.async_copy(uemb.at[uidx_v], urows_v, usem)
        icopy = pltpu.async_copy(iemb.at[iidx_v], irows_v, isem)
        ucopy.wait()
        pltpu.sync_copy(urows_v, wu_out.at[pl.ds(base, B_PER_W)])
        icopy.wait()
        pltpu.sync_copy(irows_v, hi_out.at[pl.ds(base, B_PER_W)])

    return gather_kernel(user_emb, item_emb, user_idx, item_idx)


BM = 512


def _mm_body(w_ref, h_ref, o_ref):
    o_ref[...] = lax.dot_general(
        w_ref[...], h_ref[...],
        (((1,), (1,)), ((), ())),
        preferred_element_type=jnp.float32,
    )


def _tc_matmul(w_u, h_i):
    return pl.pallas_call(
        _mm_body,
        grid=(B // BM,),
        in_specs=[
            pl.BlockSpec((BM, D), lambda i: (i, 0)),
            pl.BlockSpec((B, D), lambda i: (0, 0)),
        ],
        out_specs=pl.BlockSpec((BM, B), lambda i: (i, 0)),
        out_shape=jax.ShapeDtypeStruct((B, B), jnp.float32),
    )(w_u, h_i)


def kernel(user_tensor, item_tensor, user_emb, item_emb):
    w_u = jnp.take(user_emb, user_tensor, axis=0)
    h_i = jnp.take(item_emb, item_tensor, axis=0)
    return _tc_matmul(w_u, h_i)
